# Initial kernel scaffold; baseline (speedup 1.0000x reference)
import jax
import jax.numpy as jnp
from jax.experimental import pallas as pl


def kernel(x, edge_index, W_lin, attn_src, attn_dst, bias, W_fin, b_fin):
    raise NotImplementedError("write your pallas kernel here")


# speedup vs baseline: 74.8977x; 74.8977x over previous
import functools

import numpy as np
import jax
import jax.numpy as jnp
from jax import lax
from jax.experimental import pallas as pl
from jax.experimental.pallas import tpu as pltpu
from jax.experimental.pallas import tpu_sc as plsc

N = 10000
NPAD = 10240
IN_DIM = 128
H = 4
D = 32
HD = H * D
OUT_DIM = 128

NW = 32
K = 128
CH_PER_W = 159
EPW = CH_PER_W * K
EPAD = NW * EPW
E2 = 2 * 320000 + N

_ROWS_PER_SUB = NPAD // 16
_COPIES = _ROWS_PER_SUB // K

_S4 = np.zeros((4, HD), np.float32)
for _h in range(4):
    _S4[_h, _h * D:(_h + 1) * D] = 1.0


def _stage1_body(x_ref, wlt_ref, mcat_ref, proj_ref, alph_ref):
    xb = x_ref[...]
    proj = jnp.dot(xb, wlt_ref[...], preferred_element_type=jnp.float32)
    proj_ref[...] = proj
    alph_ref[...] = jnp.dot(proj, mcat_ref[...],
                            preferred_element_type=jnp.float32)


def _stage1(xpad, wlt, mcat):
    B = 1024
    grid = NPAD // B
    return pl.pallas_call(
        _stage1_body,
        grid=(grid,),
        in_specs=[
            pl.BlockSpec((B, IN_DIM), lambda i: (i, 0)),
            pl.BlockSpec((IN_DIM, HD), lambda i: (0, 0)),
            pl.BlockSpec((IN_DIM, 16), lambda i: (0, 0)),
        ],
        out_specs=[
            pl.BlockSpec((B, HD), lambda i: (i, 0)),
            pl.BlockSpec((B, 16), lambda i: (i, 0)),
        ],
        out_shape=[
            jax.ShapeDtypeStruct((NPAD, HD), jnp.float32),
            jax.ShapeDtypeStruct((NPAD, 16), jnp.float32),
        ],
    )(xpad, wlt, mcat)


_sc_mesh = plsc.VectorSubcoreMesh(core_axis_name="c", subcore_axis_name="s")
_sc_params = pltpu.CompilerParams(needs_layout_passes=False)


@functools.partial(
    pl.kernel,
    out_type=[
        jax.ShapeDtypeStruct((EPAD * 4,), jnp.float32),
        jax.ShapeDtypeStruct((2, 16, H, NPAD), jnp.float32),
    ],
    mesh=_sc_mesh,
    compiler_params=_sc_params,
    scratch_types=[
        pltpu.VMEM((4 * NPAD,), jnp.float32),
        pltpu.VMEM((4 * NPAD,), jnp.float32),
        pltpu.VMEM((H, NPAD), jnp.float32),
        pltpu.VMEM((K,), jnp.int32),
        pltpu.VMEM((K,), jnp.int32),
        pltpu.VMEM((K * 4,), jnp.float32),
    ],
)
def _score_kernel(asrc_hbm, adst_hbm, src_hbm, dst_hbm,
                  w_hbm, denout_hbm,
                  asrc_v, adst_v, den_v, src_v, dst_v, w_v):
    c = lax.axis_index("c")
    s = lax.axis_index("s")
    wid = s * 2 + c

    pltpu.sync_copy(asrc_hbm, asrc_v)
    pltpu.sync_copy(adst_hbm, adst_v)

    zero16 = jnp.zeros((16,), jnp.float32)

    def _zden(i, carry):
        for h in range(H):
            den_v[h, pl.ds(i * 16, 16)] = zero16
        return carry

    lax.fori_loop(0, NPAD // 16, _zden, 0)

    iota16 = lax.iota(jnp.int32, 16)

    def _chunk(ch, carry):
        base = wid * EPW + ch * K
        pltpu.sync_copy(src_hbm.at[pl.ds(base, K)], src_v)
        pltpu.sync_copy(dst_hbm.at[pl.ds(base, K)], dst_v)
        for g in range(K // 16):
            src16 = src_v[pl.ds(g * 16, 16)]
            dst16 = dst_v[pl.ds(g * 16, 16)]
            e16 = g * 16 + iota16
            for h in range(H):
                a = plsc.load_gather(asrc_v, [src16 * 4 + h])
                b = plsc.load_gather(adst_v, [dst16 * 4 + h])
                sc = a + b
                sc = jnp.where(sc >= 0.0, sc, 0.2 * sc)
                w = jnp.exp(sc)
                plsc.store_scatter(w_v, [e16 * 4 + h], w)
                plsc.addupdate_scatter(
                    den_v, [jnp.full((16,), h, jnp.int32), dst16], w)
        pltpu.sync_copy(w_v, w_hbm.at[pl.ds(base * 4, K * 4)])
        return carry

    lax.fori_loop(0, CH_PER_W, _chunk, 0)
    pltpu.sync_copy(den_v, denout_hbm.at[c, s])


@functools.partial(
    pl.kernel,
    out_type=jax.ShapeDtypeStruct((2, NPAD, HD), jnp.float32),
    mesh=_sc_mesh,
    compiler_params=_sc_params,
    scratch_types=[
        pltpu.VMEM((K,), jnp.int32),
        pltpu.VMEM((K,), jnp.int32),
        pltpu.VMEM((K * 4 + 16,), jnp.float32),
        pltpu.VMEM((K, HD), jnp.float32),
        pltpu.VMEM((K, HD), jnp.float32),
        pltpu.VMEM_SHARED((NPAD, HD), jnp.float32),
        pltpu.SemaphoreType.DMA,
    ],
)
def _message_kernel(proj_hbm, src_hbm, dst_hbm, w_hbm, accout_hbm,
                    src_v, dst_v, w_v, rows_v, out_v, acc, sem):
    c = lax.axis_index("c")
    s = lax.axis_index("s")
    wid = s * 2 + c

    zero16 = jnp.zeros((16,), jnp.float32)

    def _zout(r, carry):
        for j in range(HD // 16):
            out_v[r, pl.ds(j * 16, 16)] = zero16
        return carry

    lax.fori_loop(0, K, _zout, 0)
    for j in range(_COPIES):
        pltpu.sync_copy(out_v, acc.at[pl.ds(s * _ROWS_PER_SUB + j * K, K), :])
    plsc.subcore_barrier()

    def _chunk(ch, carry):
        base = wid * EPW + ch * K
        pltpu.sync_copy(src_hbm.at[pl.ds(base, K)], src_v)
        pltpu.sync_copy(dst_hbm.at[pl.ds(base, K)], dst_v)
        pltpu.sync_copy(w_hbm.at[pl.ds(base * 4, K * 4)],
                        w_v.at[pl.ds(0, K * 4)])
        pltpu.async_copy(proj_hbm.at[src_v], rows_v, sem).wait()

        def _edge(e, carry2):
            wvec = w_v[pl.ds(e * 4, 16)]
            for h in range(H):
                wv = wvec[h]
                for j in range(2):
                    off = h * D + j * 16
                    out_v[e, pl.ds(off, 16)] = rows_v[e, pl.ds(off, 16)] * wv
            return carry2

        lax.fori_loop(0, K, _edge, 0)
        pltpu.sync_copy(out_v, acc.at[dst_v], add=True)
        return carry

    lax.fori_loop(0, CH_PER_W, _chunk, 0)
    plsc.subcore_barrier()
    for j in range(_COPIES):
        rs = s * _ROWS_PER_SUB + j * K
        pltpu.sync_copy(acc.at[pl.ds(rs, K), :],
                        accout_hbm.at[c, pl.ds(rs, K), :])


def _stage3_body(acc_ref, den_ref, s4_ref, bias_ref, wft_ref, bfin_ref,
                 out_ref):
    m = acc_ref[0] + acc_ref[1]
    denp = jnp.sum(den_ref[...], axis=0)
    den = lax.dot_general(denp, s4_ref[...], (((0,), (0,)), ((), ())),
                          preferred_element_type=jnp.float32)
    y = m / den + bias_ref[...]
    y = jnp.where(y > 0.0, y, jnp.exp(y) - 1.0)
    out_ref[...] = jnp.dot(y, wft_ref[...],
                           preferred_element_type=jnp.float32) + bfin_ref[...]


def _stage3(accout, den3, s4, bias2, wft, bfin2):
    B = 1024
    grid = NPAD // B
    return pl.pallas_call(
        _stage3_body,
        grid=(grid,),
        in_specs=[
            pl.BlockSpec((2, B, HD), lambda i: (0, i, 0)),
            pl.BlockSpec((NW, H, B), lambda i: (0, 0, i)),
            pl.BlockSpec((H, HD), lambda i: (0, 0)),
            pl.BlockSpec((1, HD), lambda i: (0, 0)),
            pl.BlockSpec((HD, OUT_DIM), lambda i: (0, 0)),
            pl.BlockSpec((1, OUT_DIM), lambda i: (0, 0)),
        ],
        out_specs=pl.BlockSpec((B, OUT_DIM), lambda i: (i, 0)),
        out_shape=jax.ShapeDtypeStruct((NPAD, OUT_DIM), jnp.float32),
    )(accout, den3, s4, bias2, wft, bfin2)


def kernel(x, edge_index, W_lin, attn_src, attn_dst, bias, W_fin, b_fin):
    xpad = jnp.pad(x, ((0, NPAD - N), (0, 0)))
    wlt = W_lin.T
    s4 = jnp.asarray(_S4)
    mcat = jnp.concatenate(
        [s4.T * attn_src.reshape(-1)[:, None],
         s4.T * attn_dst.reshape(-1)[:, None],
         jnp.zeros((HD, 8), jnp.float32)], axis=1)

    proj, alph = _stage1(xpad, wlt, mcat)
    asrc_flat = alph[:, 0:4].reshape(-1)
    adst_flat = alph[:, 4:8].reshape(-1)

    src0 = edge_index[0]
    dst0 = edge_index[1]
    loop = jnp.arange(N, dtype=jnp.int32)
    padlen = EPAD - E2
    src = jnp.concatenate(
        [src0, dst0, loop, jnp.zeros((padlen,), jnp.int32)])
    dst = jnp.concatenate(
        [dst0, src0, loop, jnp.full((padlen,), N, jnp.int32)])

    w_flat, denout = _score_kernel(asrc_flat, adst_flat, src, dst)
    accout = _message_kernel(proj, src, dst, w_flat)
    den3 = denout.reshape(NW, H, NPAD)

    out = _stage3(accout, den3, s4, bias.reshape(1, HD),
                  W_fin.T, b_fin.reshape(1, OUT_DIM))
    return out[:N]
